# baseline (speedup 1.0000x reference)
import functools
import math

import jax
import jax.numpy as jnp
from jax import lax
from jax.experimental import pallas as pl
from jax.experimental.pallas import tpu as pltpu
from jax.experimental.pallas import tpu_sc as plsc

_D = 64
_NC, _NS = 2, 16
_NW = _NC * _NS
_LANES = 16
_SCALE = math.sqrt(_D)


@functools.partial(jax.jit, static_argnames=("B", "C"))
def _embed_lookup(tokens_eo, table, *, B, C):
    b_per_w = B // _NW
    nchunks = b_per_w // C
    assert nchunks % 2 == 0
    H = C // 2
    mesh = plsc.VectorSubcoreMesh(
        core_axis_name="c", subcore_axis_name="s",
        num_cores=_NC, num_subcores=_NS)

    @functools.partial(
        pl.kernel,
        out_type=jax.ShapeDtypeStruct((B // 2, 2 * _D), jnp.float32),
        mesh=mesh,
        compiler_params=pltpu.CompilerParams(use_tc_tiling_on_sc=False),
        scratch_types=[
            pltpu.VMEM((2, 2, H), jnp.int32),
            pltpu.VMEM((2, 2, H, _D), jnp.float32),
            pltpu.SemaphoreType.DMA,
            pltpu.SemaphoreType.DMA,
            pltpu.SemaphoreType.DMA,
            pltpu.SemaphoreType.DMA,
        ],
    )
    def k(tokens_hbm, table_hbm, out_hbm, idx_v, rows_v, gs0, gs1, os0, os1):
        gsems = (gs0, gs1)
        osems = (os0, os1)
        wid = lax.axis_index("s") * _NC + lax.axis_index("c")
        base = wid * (b_per_w // 2)

        def start_gather(buf, g, gsem):
            off = base + g * H
            pltpu.sync_copy(tokens_hbm.at[0, pl.ds(off, H)], idx_v.at[buf, 0])
            pltpu.sync_copy(tokens_hbm.at[1, pl.ds(off, H)], idx_v.at[buf, 1])
            pltpu.async_copy(
                table_hbm.at[idx_v.at[buf, 0]], rows_v.at[buf, 0], gsem)
            pltpu.async_copy(
                table_hbm.at[idx_v.at[buf, 1]], rows_v.at[buf, 1], gsem)

        def wait_gather(buf, gsem):
            pltpu.make_async_copy(
                table_hbm.at[idx_v.at[buf, 0]], rows_v.at[buf, 0], gsem).wait()
            pltpu.make_async_copy(
                table_hbm.at[idx_v.at[buf, 1]], rows_v.at[buf, 1], gsem).wait()

        start_gather(0, 0, gs0)

        @pl.loop(0, nchunks, step=2)
        def _outer(G):
            for b in range(2):
                g = G + b
                nb = 1 - b

                @pl.when(g + 1 < nchunks)
                def _start_next():
                    @pl.when(g >= 1)
                    def _drain():
                        for eo in range(2):
                            pltpu.make_async_copy(
                                rows_v.at[nb, eo],
                                out_hbm.at[pl.ds(0, H), pl.ds(eo * _D, _D)],
                                osems[nb]).wait()
                    start_gather(nb, g + 1, gsems[nb])

                wait_gather(b, gsems[b])

                @pl.loop(0, H, unroll=4)
                def _scale(r):
                    for eo in range(2):
                        for j in range(_D // _LANES):
                            sl = pl.ds(j * _LANES, _LANES)
                            rows_v[b, eo, r, sl] = rows_v[b, eo, r, sl] * _SCALE

                for eo in range(2):
                    pltpu.async_copy(
                        rows_v.at[b, eo],
                        out_hbm.at[pl.ds(base + g * H, H),
                                   pl.ds(eo * _D, _D)],
                        osems[b])

        for b, osem in ((0, os0), (1, os1)):
            for eo in range(2):
                pltpu.make_async_copy(
                    rows_v.at[b, eo],
                    out_hbm.at[pl.ds(0, H), pl.ds(eo * _D, _D)],
                    osem).wait()

    return k(tokens_eo, table)


def kernel(tokens, table):
    B = tokens.shape[0] * tokens.shape[1]
    tokens_eo = tokens.reshape(B // 2, 2).astype(jnp.int32).T
    out = _embed_lookup(tokens_eo, table, B=B, C=512)
    return out.reshape(tokens.shape[0], tokens.shape[1], _D)
